# baseline (speedup 1.0000x reference)
import functools

import jax
import jax.numpy as jnp
from jax import lax
from jax.experimental import pallas as pl
from jax.experimental.pallas import tpu as pltpu
from jax.experimental.pallas import tpu_sc as plsc

B = 16384
D = 64
EPS = 1e-5

_NC = 2
_NS = 16
_NW = _NC * _NS
_BPW = B // _NW


def _gather_body(uid_hbm, iid_hbm, ut_hbm, it_hbm, uout, iout,
                 uidx_v, urows_v, iidx_v, irows_v, usem, isem):
    wid = lax.axis_index("s") * _NC + lax.axis_index("c")
    base = wid * _BPW
    pltpu.sync_copy(uid_hbm.at[pl.ds(base, _BPW)], uidx_v)
    pltpu.sync_copy(iid_hbm.at[pl.ds(base, _BPW)], iidx_v)
    cu = pltpu.async_copy(ut_hbm.at[uidx_v], urows_v, usem)
    ci = pltpu.async_copy(it_hbm.at[iidx_v], irows_v, isem)
    cu.wait()
    ci.wait()
    pltpu.sync_copy(urows_v, uout.at[pl.ds(base, _BPW)])
    pltpu.sync_copy(irows_v, iout.at[pl.ds(base, _BPW)])


@jax.jit
def _sc_gather(user_ids, item_ids, user_table, item_table):
    mesh = plsc.VectorSubcoreMesh(core_axis_name="c", subcore_axis_name="s")
    f = pl.kernel(
        _gather_body,
        out_type=[
            jax.ShapeDtypeStruct((B, D), jnp.float32),
            jax.ShapeDtypeStruct((B, D), jnp.float32),
        ],
        mesh=mesh,
        scratch_types=[
            pltpu.VMEM((_BPW,), jnp.int32),
            pltpu.VMEM((_BPW, D), jnp.float32),
            pltpu.VMEM((_BPW,), jnp.int32),
            pltpu.VMEM((_BPW, D), jnp.float32),
            pltpu.SemaphoreType.DMA,
            pltpu.SemaphoreType.DMA,
        ],
        compiler_params=pltpu.CompilerParams(use_tc_tiling_on_sc=False),
    )
    return f(user_ids, item_ids, user_table, item_table)


def _mlp_body(x_ref, w1_ref, b1_ref, g1_ref, be1_ref,
              w2_ref, b2_ref, g2_ref, be2_ref, w3_ref, b3_ref, out_ref):
    x = x_ref[...]
    h = jnp.dot(x, w1_ref[...], preferred_element_type=jnp.float32) + b1_ref[...]
    mean = jnp.mean(h, axis=0, keepdims=True)
    var = jnp.mean(jnp.square(h - mean), axis=0, keepdims=True)
    h = (h - mean) * lax.rsqrt(var + EPS) * g1_ref[...] + be1_ref[...]
    h = jnp.maximum(h, 0.0)
    h = jnp.dot(h, w2_ref[...], preferred_element_type=jnp.float32) + b2_ref[...]
    mean = jnp.mean(h, axis=0, keepdims=True)
    var = jnp.mean(jnp.square(h - mean), axis=0, keepdims=True)
    h = (h - mean) * lax.rsqrt(var + EPS) * g2_ref[...] + be2_ref[...]
    h = jnp.maximum(h, 0.0)
    o = jnp.dot(h, w3_ref[...], preferred_element_type=jnp.float32) + b3_ref[...]
    out_ref[...] = jax.nn.sigmoid(o)


@jax.jit
def _tc_mlp(x, W1, b1, g1, be1, W2, b2, g2, be2, W3, b3):
    return pl.pallas_call(
        _mlp_body,
        out_shape=jax.ShapeDtypeStruct((B, 1), jnp.float32),
    )(x, W1, b1.reshape(1, -1), g1.reshape(1, -1), be1.reshape(1, -1),
      W2, b2.reshape(1, -1), g2.reshape(1, -1), be2.reshape(1, -1),
      W3, b3.reshape(1, -1))


def kernel(user_ids, item_ids, user_table, item_table,
           W1, b1, g1, be1, W2, b2, g2, be2, W3, b3):
    user_emb, item_emb = _sc_gather(
        user_ids.astype(jnp.int32), item_ids.astype(jnp.int32),
        user_table, item_table)
    x = jnp.concatenate([user_emb, item_emb], axis=1)
    out = _tc_mlp(x, W1, b1, g1, be1, W2, b2, g2, be2, W3, b3)
    return jnp.squeeze(out, axis=-1)


# speedup vs baseline: 1.5752x; 1.5752x over previous
import functools

import jax
import jax.numpy as jnp
from jax import lax
from jax.experimental import pallas as pl
from jax.experimental.pallas import tpu as pltpu
from jax.experimental.pallas import tpu_sc as plsc

B = 16384
D = 64
EPS = 1e-5

_NC = 2
_NS = 16
_NW = _NC * _NS
_BPW = B // _NW


def _gather_one(idx_v, table_hbm, rows_v, out_hbm, base, sem):
    def body(g, carry):
        vec = idx_v[pl.ds(g * 16, 16)]
        for j in range(16):
            r = vec[j]
            pltpu.async_copy(
                table_hbm.at[pl.ds(r, 1), :],
                rows_v.at[pl.ds(g * 16 + j, 1), :], sem)
        return carry

    lax.fori_loop(0, _BPW // 16, body, 0)
    pltpu.make_async_copy(table_hbm.at[pl.ds(0, _BPW), :], rows_v, sem).wait()
    pltpu.sync_copy(rows_v, out_hbm.at[pl.ds(base, _BPW)])


def _gather_body(uid_hbm, iid_hbm, ut_hbm, it_hbm, uout, iout,
                 uidx_v, iidx_v, rows_v, sem):
    wid = lax.axis_index("s") * _NC + lax.axis_index("c")
    base = wid * _BPW
    pltpu.sync_copy(uid_hbm.at[pl.ds(base, _BPW)], uidx_v)
    pltpu.sync_copy(iid_hbm.at[pl.ds(base, _BPW)], iidx_v)
    _gather_one(uidx_v, ut_hbm, rows_v, uout, base, sem)
    _gather_one(iidx_v, it_hbm, rows_v, iout, base, sem)


@jax.jit
def _sc_gather(user_ids, item_ids, user_table, item_table):
    mesh = plsc.VectorSubcoreMesh(core_axis_name="c", subcore_axis_name="s")
    f = pl.kernel(
        _gather_body,
        out_type=[
            jax.ShapeDtypeStruct((B, D), jnp.float32),
            jax.ShapeDtypeStruct((B, D), jnp.float32),
        ],
        mesh=mesh,
        scratch_types=[
            pltpu.VMEM((_BPW,), jnp.int32),
            pltpu.VMEM((_BPW,), jnp.int32),
            pltpu.VMEM((_BPW, D), jnp.float32),
            pltpu.SemaphoreType.DMA,
        ],
    )
    return f(user_ids, item_ids, user_table, item_table)


def _mlp_body(ue_ref, ie_ref, w1a_ref, w1b_ref, b1_ref, g1_ref, be1_ref,
              w2_ref, b2_ref, g2_ref, be2_ref, w3_ref, b3_ref, out_ref):
    h = (jnp.dot(ue_ref[...], w1a_ref[...], preferred_element_type=jnp.float32)
         + jnp.dot(ie_ref[...], w1b_ref[...], preferred_element_type=jnp.float32)
         + b1_ref[...])
    mean = jnp.mean(h, axis=0, keepdims=True)
    var = jnp.mean(jnp.square(h - mean), axis=0, keepdims=True)
    h = (h - mean) * lax.rsqrt(var + EPS) * g1_ref[...] + be1_ref[...]
    h = jnp.maximum(h, 0.0)
    h = jnp.dot(h, w2_ref[...], preferred_element_type=jnp.float32) + b2_ref[...]
    mean = jnp.mean(h, axis=0, keepdims=True)
    var = jnp.mean(jnp.square(h - mean), axis=0, keepdims=True)
    h = (h - mean) * lax.rsqrt(var + EPS) * g2_ref[...] + be2_ref[...]
    h = jnp.maximum(h, 0.0)
    o = jnp.dot(h, w3_ref[...], preferred_element_type=jnp.float32) + b3_ref[...]
    out_ref[...] = jax.nn.sigmoid(o)


@jax.jit
def _tc_mlp(ue, ie, W1, b1, g1, be1, W2, b2, g2, be2, W3, b3):
    return pl.pallas_call(
        _mlp_body,
        out_shape=jax.ShapeDtypeStruct((B, 1), jnp.float32),
    )(ue, ie, W1[:D], W1[D:], b1.reshape(1, -1), g1.reshape(1, -1),
      be1.reshape(1, -1), W2, b2.reshape(1, -1), g2.reshape(1, -1),
      be2.reshape(1, -1), W3, b3.reshape(1, -1))


def kernel(user_ids, item_ids, user_table, item_table,
           W1, b1, g1, be1, W2, b2, g2, be2, W3, b3):
    user_emb, item_emb = _sc_gather(
        user_ids.astype(jnp.int32), item_ids.astype(jnp.int32),
        user_table, item_table)
    out = _tc_mlp(user_emb, item_emb,
                  W1, b1, g1, be1, W2, b2, g2, be2, W3, b3)
    return jnp.squeeze(out, axis=-1)


# speedup vs baseline: 1.6057x; 1.0194x over previous
import jax
import jax.numpy as jnp
from jax import lax
from jax.experimental import pallas as pl
from jax.experimental.pallas import tpu as pltpu
from jax.experimental.pallas import tpu_sc as plsc

B = 16384
D = 64
N_ROWS = 1000000
EPS = 1e-5

_BL = 2048
_NBLK = 245
_HALF0 = _NBLK * _BL
_LASTBLK = (N_ROWS + _BL - 1) // _BL - 1

_NC = 2
_NS = 16
_NW = _NC * _NS
_BPW = B // _NW
_CH = 128
_NCH = _BPW // _CH


def _relayout_body(x1_ref, x2_ref, out_ref):
    xt1 = jnp.transpose(x1_ref[...])
    xt2 = jnp.transpose(x2_ref[...])
    out_ref[...] = jnp.concatenate([xt1, xt2], axis=1)


def _relayout(tt):
    return pl.pallas_call(
        _relayout_body,
        grid=(_NBLK,),
        in_specs=[
            pl.BlockSpec((D, _BL), lambda i: (0, i)),
            pl.BlockSpec((D, _BL), lambda i: (0, jnp.minimum(i + _NBLK, _LASTBLK))),
        ],
        out_specs=pl.BlockSpec((_BL, 2 * D), lambda i: (i, 0)),
        out_shape=jax.ShapeDtypeStruct((_HALF0, 2 * D), jnp.float32),
    )(tt, tt)


def _gather_one(id_hbm, c_hbm, out_hbm, base, idx_v, fidx_v, rows_v, sem):
    for c in range(_NCH):
        pltpu.sync_copy(id_hbm.at[pl.ds(base + c * _CH, _CH)], idx_v.at[c])
    for c in range(_NCH):
        for g in range(_CH // 16):
            v = idx_v[c, pl.ds(g * 16, 16)]
            fidx_v[c, pl.ds(g * 16, 16)] = jnp.where(v < _HALF0, v, v - _HALF0)
    for c in range(_NCH):
        pltpu.async_copy(c_hbm.at[fidx_v.at[c]],
                         rows_v.at[pl.ds(c * _CH, _CH)], sem)
    pltpu.make_async_copy(c_hbm.at[pl.ds(0, _BPW)], rows_v, sem).wait()
    pltpu.sync_copy(rows_v, out_hbm.at[pl.ds(base, _BPW)])


def _gather_body(uid_hbm, iid_hbm, uc_hbm, ic_hbm, uout, iout,
                 idx_v, fidx_v, rows_v, sem):
    wid = lax.axis_index("s") * _NC + lax.axis_index("c")
    base = wid * _BPW
    _gather_one(uid_hbm, uc_hbm, uout, base, idx_v, fidx_v, rows_v, sem)
    _gather_one(iid_hbm, ic_hbm, iout, base, idx_v, fidx_v, rows_v, sem)


def _sc_gather(user_ids, item_ids, uc, ic):
    mesh = plsc.VectorSubcoreMesh(core_axis_name="c", subcore_axis_name="s")
    f = pl.kernel(
        _gather_body,
        out_type=[
            jax.ShapeDtypeStruct((B, 2 * D), jnp.float32),
            jax.ShapeDtypeStruct((B, 2 * D), jnp.float32),
        ],
        mesh=mesh,
        scratch_types=[
            pltpu.VMEM((_NCH, _CH), jnp.int32),
            pltpu.VMEM((_NCH, _CH), jnp.int32),
            pltpu.VMEM((_BPW, 2 * D), jnp.float32),
            pltpu.SemaphoreType.DMA,
        ],
    )
    return f(user_ids, item_ids, uc, ic)


def _mlp_body(gu_ref, gi_ref, mu_ref, mi_ref, w1a_ref, w1b_ref, b1_ref,
              g1_ref, be1_ref, w2_ref, b2_ref, g2_ref, be2_ref,
              w3_ref, b3_ref, out_ref):
    gu = gu_ref[...] * mu_ref[...]
    gi = gi_ref[...] * mi_ref[...]
    ue = gu[:, :D] + gu[:, D:]
    ie = gi[:, :D] + gi[:, D:]
    h = (jnp.dot(ue, w1a_ref[...], preferred_element_type=jnp.float32)
         + jnp.dot(ie, w1b_ref[...], preferred_element_type=jnp.float32)
         + b1_ref[...])
    mean = jnp.mean(h, axis=0, keepdims=True)
    var = jnp.mean(jnp.square(h - mean), axis=0, keepdims=True)
    h = (h - mean) * lax.rsqrt(var + EPS) * g1_ref[...] + be1_ref[...]
    h = jnp.maximum(h, 0.0)
    h = jnp.dot(h, w2_ref[...], preferred_element_type=jnp.float32) + b2_ref[...]
    mean = jnp.mean(h, axis=0, keepdims=True)
    var = jnp.mean(jnp.square(h - mean), axis=0, keepdims=True)
    h = (h - mean) * lax.rsqrt(var + EPS) * g2_ref[...] + be2_ref[...]
    h = jnp.maximum(h, 0.0)
    o = jnp.dot(h, w3_ref[...], preferred_element_type=jnp.float32) + b3_ref[...]
    out_ref[...] = jax.nn.sigmoid(o)


def _tc_mlp(gu, gi, mu, mi, W1, b1, g1, be1, W2, b2, g2, be2, W3, b3):
    return pl.pallas_call(
        _mlp_body,
        out_shape=jax.ShapeDtypeStruct((B, 1), jnp.float32),
    )(gu, gi, mu, mi, W1[:D], W1[D:], b1.reshape(1, -1), g1.reshape(1, -1),
      be1.reshape(1, -1), W2, b2.reshape(1, -1), g2.reshape(1, -1),
      be2.reshape(1, -1), W3, b3.reshape(1, -1))


def _half_mask(ids):
    hi = (ids >= _HALF0)[:, None]
    lane_hi = (jnp.arange(2 * D, dtype=jnp.int32) >= D)[None, :]
    return (hi == lane_hi).astype(jnp.float32)


def kernel(user_ids, item_ids, user_table, item_table,
           W1, b1, g1, be1, W2, b2, g2, be2, W3, b3):
    uids = user_ids.astype(jnp.int32)
    iids = item_ids.astype(jnp.int32)
    cu = _relayout(user_table.T)
    ci = _relayout(item_table.T)
    gu, gi = _sc_gather(uids, iids, cu, ci)
    out = _tc_mlp(gu, gi, _half_mask(uids), _half_mask(iids),
                  W1, b1, g1, be1, W2, b2, g2, be2, W3, b3)
    return jnp.squeeze(out, axis=-1)
